# baseline (speedup 1.0000x reference)
import functools

import jax
import jax.numpy as jnp
from jax.experimental import pallas as pl
from jax.experimental.pallas import tpu as pltpu


def _bdot(a, b):
    return jnp.dot(a, b, preferred_element_type=jnp.float32)


def _gcn_body(nb, bm, bmb,
              x_ref, w1_ref, b1_ref, w2t_ref, b2_ref, w3_ref, b3_ref,
              adj_ref,
              lsm_ref, out_ref,
              adjt_scr, xw1t_scr, hw2t_scr):
    i = pl.program_id(0)

    @pl.when(i == 0)
    def _init():
        xw1 = _bdot(x_ref[...], w1_ref[...])
        xw1t_scr[...] = xw1.T.astype(jnp.bfloat16)

    @pl.when(i < nb)
    def _stream():
        half = bm // 2
        for c in range(2):
            abt = adj_ref[c * half:(c + 1) * half, :].astype(
                jnp.bfloat16).T
            adjt_scr[:, pl.ds(i * bm + c * half, half)] = abt
            ht = jnp.maximum(
                _bdot(xw1t_scr[...], abt) + b1_ref[...], 0.0)
            hw2t_scr[:, pl.ds(i * bm + c * half, half)] = _bdot(
                w2t_ref[...], ht).astype(jnp.bfloat16)

    @pl.when(i >= nb)
    def _head():
        base = (i - nb) * bmb
        h2t = _bdot(hw2t_scr[...], adjt_scr[:, pl.ds(base, bmb)])
        h2 = h2t.T + b2_ref[...]
        m = jnp.max(h2, axis=1, keepdims=True)
        lse = jnp.log(jnp.sum(jnp.exp(h2 - m), axis=1, keepdims=True))
        lsm_ref[...] = (h2 - m) - lse
        r = jnp.maximum(h2, 0.0)
        out_ref[...] = _bdot(r, w3_ref[...]) + b3_ref[...]


def kernel(x, adj, W1, b1, W2, b2, W3, b3, encoder_type):
    n, nfeat = x.shape
    nhid = W1.shape[1]
    nclass = W2.shape[1]
    proj = W3.shape[1]
    del encoder_type

    bm = 512
    nb = n // bm
    bmb = 2048
    nbb = n // bmb

    b1c = b1.reshape(nhid, 1)
    b2r = b2.reshape(1, nclass)
    b3r = b3.reshape(1, proj)
    W2t = W2.T

    body = functools.partial(_gcn_body, nb, bm, bmb)

    lsm, out = pl.pallas_call(
        body,
        grid=(nb + nbb,),
        in_specs=[
            pl.BlockSpec((n, nfeat), lambda i: (0, 0)),
            pl.BlockSpec((nfeat, nhid), lambda i: (0, 0)),
            pl.BlockSpec((nhid, 1), lambda i: (0, 0)),
            pl.BlockSpec((nclass, nhid), lambda i: (0, 0)),
            pl.BlockSpec((1, nclass), lambda i: (0, 0)),
            pl.BlockSpec((nclass, proj), lambda i: (0, 0)),
            pl.BlockSpec((1, proj), lambda i: (0, 0)),
            pl.BlockSpec((bm, n),
                         lambda i: (jnp.minimum(i, nb - 1), 0)),
        ],
        out_specs=[
            pl.BlockSpec((bmb, nclass),
                         lambda i: (jnp.maximum(i - nb, 0), 0)),
            pl.BlockSpec((bmb, proj),
                         lambda i: (jnp.maximum(i - nb, 0), 0)),
        ],
        out_shape=[
            jax.ShapeDtypeStruct((n, nclass), jnp.float32),
            jax.ShapeDtypeStruct((n, proj), jnp.float32),
        ],
        scratch_shapes=[
            pltpu.VMEM((n, n), jnp.bfloat16),
            pltpu.VMEM((nhid, n), jnp.bfloat16),
            pltpu.VMEM((nclass, n), jnp.bfloat16),
        ],
        compiler_params=pltpu.CompilerParams(
            dimension_semantics=("arbitrary",),
            vmem_limit_bytes=100 * 1024 * 1024,
        ),
    )(x, W1, b1c, W2t, b2r, W3, b3r, adj)

    return (lsm, out)


# speedup vs baseline: 1.1531x; 1.1531x over previous
import functools

import jax
import jax.numpy as jnp
from jax.experimental import pallas as pl
from jax.experimental.pallas import tpu as pltpu


def _bdot(a, b):
    return jnp.dot(a, b, preferred_element_type=jnp.float32)


def _gcn_body(nb, bm, bmb,
              x_ref, w1_ref, b1_ref, w2t_ref, b2_ref, w3_ref, b3_ref,
              adj_ref,
              lsm_ref, out_ref,
              adjt_scr, xw1t_scr, hw2t_scr):
    i = pl.program_id(0)

    @pl.when(i == 0)
    def _init():
        xw1 = _bdot(x_ref[...], w1_ref[...])
        xw1t_scr[...] = xw1.T.astype(jnp.bfloat16)

    @pl.when(i < nb)
    def _stream():
        abt = adj_ref[...].astype(jnp.bfloat16).T
        adjt_scr[:, pl.ds(i * bm, bm)] = abt
        ht = jnp.maximum(_bdot(xw1t_scr[...], abt) + b1_ref[...], 0.0)
        hw2t_scr[:, pl.ds(i * bm, bm)] = _bdot(
            w2t_ref[...], ht).astype(jnp.bfloat16)

    @pl.when(i >= nb)
    def _head():
        base = (i - nb) * bmb
        h2t = _bdot(hw2t_scr[...], adjt_scr[:, pl.ds(base, bmb)])
        h2 = h2t.T + b2_ref[...]
        m = jnp.max(h2, axis=1, keepdims=True)
        lse = jnp.log(jnp.sum(jnp.exp(h2 - m), axis=1, keepdims=True))
        lsm_ref[...] = (h2 - m) - lse
        r = jnp.maximum(h2, 0.0)
        out_ref[...] = _bdot(r, w3_ref[...]) + b3_ref[...]


def kernel(x, adj, W1, b1, W2, b2, W3, b3, encoder_type):
    n, nfeat = x.shape
    nhid = W1.shape[1]
    nclass = W2.shape[1]
    proj = W3.shape[1]
    del encoder_type

    bm = 512
    nb = n // bm
    bmb = 2048
    nbb = n // bmb

    b1c = b1.reshape(nhid, 1)
    b2r = b2.reshape(1, nclass)
    b3r = b3.reshape(1, proj)
    W2t = W2.T

    body = functools.partial(_gcn_body, nb, bm, bmb)

    lsm, out = pl.pallas_call(
        body,
        grid=(nb + nbb,),
        in_specs=[
            pl.BlockSpec((n, nfeat), lambda i: (0, 0)),
            pl.BlockSpec((nfeat, nhid), lambda i: (0, 0)),
            pl.BlockSpec((nhid, 1), lambda i: (0, 0)),
            pl.BlockSpec((nclass, nhid), lambda i: (0, 0)),
            pl.BlockSpec((1, nclass), lambda i: (0, 0)),
            pl.BlockSpec((nclass, proj), lambda i: (0, 0)),
            pl.BlockSpec((1, proj), lambda i: (0, 0)),
            pl.BlockSpec((bm, n),
                         lambda i: (jnp.minimum(i, nb - 1), 0)),
        ],
        out_specs=[
            pl.BlockSpec((bmb, nclass),
                         lambda i: (jnp.maximum(i - nb, 0), 0)),
            pl.BlockSpec((bmb, proj),
                         lambda i: (jnp.maximum(i - nb, 0), 0)),
        ],
        out_shape=[
            jax.ShapeDtypeStruct((n, nclass), jnp.float32),
            jax.ShapeDtypeStruct((n, proj), jnp.float32),
        ],
        scratch_shapes=[
            pltpu.VMEM((n, n), jnp.bfloat16),
            pltpu.VMEM((nhid, n), jnp.bfloat16),
            pltpu.VMEM((nclass, n), jnp.bfloat16),
        ],
        compiler_params=pltpu.CompilerParams(
            dimension_semantics=("arbitrary",),
            vmem_limit_bytes=100 * 1024 * 1024,
        ),
    )(x, W1, b1c, W2t, b2r, W3, b3r, adj)

    return (lsm, out)
